# baseline (speedup 1.0000x reference)
import jax
import jax.numpy as jnp
from jax import lax
from jax.experimental import pallas as pl
from jax.experimental.pallas import tpu as pltpu
from jax.experimental.pallas import tpu_sc as plsc

_DATA_LEN = 1000000
_B = 16384
_G1 = 0.9
_G2 = 0.9
_NT = 16
_CH = _B // _NT
_NR = _CH // 16
_K_FAST = 2


def _scal(w):
    lane = lax.iota(jnp.int32, 16)
    return (w + lane.astype(w.dtype) * 0.0)[0]


def _lsum(w):
    lane = lax.iota(jnp.int32, 16)
    for step in (8, 4, 2, 1):
        w = w + jnp.take(w, lane ^ step)
    return w


def _body(sn_hbm, sd_hbm, snb_hbm, sdb_hbm, idst_hbm,
          snt_hbm, sdt_hbm, yt_hbm, a_hbm, b_hbm, al_hbm,
          out_hbm,
          ids_f, ivf_f, iv_f, sv_f, gf_f, g_f,
          sn_v, sd_v, y_v, bufn, bufd, snw, sdw,
          a_v, b_v, al_v, pub, allc, psum, allp, pm_f, candv_f, cgat_f,
          spm_i, spm_f, w_spm,
          sem1, sem2, sem3, sem4, sem5, sem6):
    c = lax.axis_index("c")
    tid = lax.axis_index("s")

    @pl.when(c == 0)
    def _core0():
        lane = lax.iota(jnp.int32, 16)
        base = tid * _CH

        pltpu.sync_copy(idst_hbm.at[tid], ids_f)
        h_bn = pltpu.async_copy(snb_hbm.at[ids_f], bufn, sem1)
        h_bd = pltpu.async_copy(sdb_hbm.at[ids_f], bufd, sem2)
        h_sn = pltpu.async_copy(snt_hbm.at[tid], sn_v, sem3)
        h_sd = pltpu.async_copy(sdt_hbm.at[tid], sd_v, sem4)
        h_y = pltpu.async_copy(yt_hbm.at[tid], y_v, sem5)
        h_a = pltpu.async_copy(a_hbm, a_v, sem6)
        h_b = pltpu.async_copy(b_hbm, b_v, sem6)
        h_al = pltpu.async_copy(al_hbm, al_v, sem6)

        for r in range(_NR):
            sl = pl.ds(r * 16, 16)
            iv_f[sl] = base + r * 16 + lane
            ivf_f[sl] = (base + r * 16 + lane).astype(jnp.float32)

        pltpu.sync_copy(ivf_f, w_spm.at[ids_f])
        plsc.subcore_barrier()

        for _rr in range(_K_FAST):
            pltpu.sync_copy(w_spm.at[ids_f], gf_f)
            plsc.subcore_barrier()
            for r in range(_NR):
                sl = pl.ds(r * 16, 16)
                sv_f[sl] = jnp.maximum(ivf_f[sl], gf_f[sl])
            pltpu.sync_copy(sv_f, w_spm.at[ids_f])
            plsc.subcore_barrier()

        pltpu.sync_copy(w_spm.at[ids_f], gf_f)
        nv = jnp.zeros((16,), jnp.float32)
        for r in range(_NR):
            sl = pl.ds(r * 16, 16)
            nv = nv + jnp.where(ivf_f[sl] > gf_f[sl], 1.0, 0.0)
        for r in range(_NR):
            sl = pl.ds(r * 16, 16)
            g_f[sl] = gf_f[sl].astype(jnp.int32)

        h_sn.wait()
        h_sd.wait()
        h_sw = pltpu.async_copy(sn_hbm.at[g_f], snw, sem3)
        h_dw = pltpu.async_copy(sd_hbm.at[g_f], sdw, sem4)

        pub[...] = nv
        pltpu.sync_copy(pub, spm_i.at[tid])
        plsc.subcore_barrier()
        pltpu.sync_copy(spm_i, allc)
        tot = jnp.zeros((16,), jnp.float32)
        for t in range(_NT):
            tot = tot + allc[t]
        need_slow = _scal(_lsum(tot)) > 0.5
        h_sw.wait()
        h_dw.wait()

        @pl.when(need_slow)
        def _bit_descent():
            for r in range(_NR):
                sl = pl.ds(r * 16, 16)
                pm_f[sl] = jnp.zeros((16,), jnp.int32) + 1
                g_f[sl] = jnp.zeros((16,), jnp.int32)

            def _bit_round(bnum, carry):
                bitpos = 13 - bnum
                for r in range(_NR):
                    sl = pl.ds(r * 16, 16)
                    candv_f[sl] = jnp.zeros((16,), jnp.float32)
                pltpu.sync_copy(candv_f, w_spm.at[ids_f])
                plsc.subcore_barrier()
                for r in range(_NR):
                    sl = pl.ds(r * 16, 16)
                    bitv = lax.shift_right_logical(iv_f[sl], bitpos) & 1
                    candv_f[sl] = (pm_f[sl] & bitv).astype(jnp.float32)
                pltpu.sync_copy(candv_f, w_spm.at[ids_f], add=True)
                plsc.subcore_barrier()
                pltpu.sync_copy(w_spm.at[ids_f], cgat_f)
                for r in range(_NR):
                    sl = pl.ds(r * 16, 16)
                    mb = jnp.where(cgat_f[sl] > 0.5, 1, 0)
                    bitv = lax.shift_right_logical(iv_f[sl], bitpos) & 1
                    g_f[sl] = g_f[sl] | lax.shift_left(mb, bitpos)
                    pm_f[sl] = pm_f[sl] & jnp.where(bitv == mb, 1, 0)
                plsc.subcore_barrier()
                return carry

            lax.fori_loop(0, 14, _bit_round, jnp.int32(0))
            pltpu.async_copy(sn_hbm.at[g_f], snw, sem3).wait()
            pltpu.async_copy(sd_hbm.at[g_f], sdw, sem4).wait()

        h_y.wait()
        h_a.wait()
        h_b.wait()
        h_al.wait()
        h_bn.wait()
        h_bd.wait()

        av = a_v[...]
        bv = b_v[...]
        z = jnp.zeros((16,), jnp.float32)
        acc_gwp = z
        acc_gwn = z
        acc_ap = z
        acc_an = z
        acc_ga = z
        acc_gb = z
        acc_cp = z
        acc_cn = z
        for r in range(_NR):
            sl = pl.ds(r * 16, 16)
            vsn = (1.0 - _G1) * bufn[sl] + _G1 * snw[sl]
            vsd = (1.0 - _G2) * bufd[sl] + _G2 * sdw[sl]
            vsd = jnp.maximum(vsd, 1e-8)
            snd = 1.0 / (1.0 + jnp.exp(-(vsn / vsd)))
            gsnd = snd * (1.0 - snd)
            gw = gsnd * (sn_v[sl] / vsd - vsn / (vsd * vsd) * sd_v[sl])
            mp = jnp.where(y_v[sl] == 1, 1.0, 0.0)
            mn = 1.0 - mp
            dp = snd - av
            dn = snd - bv
            acc_gwp = acc_gwp + mp * (2.0 * dp * gw)
            acc_gwn = acc_gwn + mn * (2.0 * dn * gw)
            acc_ap = acc_ap + mp * gw
            acc_an = acc_an + mn * gw
            acc_ga = acc_ga + mp * dp * dp
            acc_gb = acc_gb + mn * dn * dn
            acc_cp = acc_cp + mp
            acc_cn = acc_cn + mn

        sums = (acc_gwp, acc_gwn, acc_ap, acc_an,
                acc_ga, acc_gb, acc_cp, acc_cn)
        vec = z
        for k, acc in enumerate(sums):
            vec = jnp.where(lane == k, _lsum(acc), vec)
        psum[...] = vec
        pltpu.sync_copy(psum, spm_f.at[tid])
        plsc.subcore_barrier()

        @pl.when(tid == 0)
        def _final():
            pltpu.sync_copy(spm_f, allp)
            tot2 = z
            for t in range(_NT):
                tot2 = tot2 + allp[t]
            s_gwp, s_gwn, s_ap, s_an, s_ga, s_gb, s_cp, s_cn = (
                jnp.take(tot2, lane * 0 + k) for k in range(8))
            alpha_v = al_v[...]
            loss = (s_gwp / s_cp + s_gwn / s_cn
                    + alpha_v * (s_an / s_cn - s_ap / s_cp)
                    + s_ga / s_cp + s_gb / s_cn)
            psum[...] = jnp.where(lane == 0, loss, 0.0)
            pltpu.sync_copy(psum, out_hbm)


_mesh = plsc.VectorSubcoreMesh(core_axis_name="c", subcore_axis_name="s")

_kern = pl.kernel(
    _body,
    out_type=jax.ShapeDtypeStruct((16,), jnp.float32),
    mesh=_mesh,
    scratch_types=[
        pltpu.VMEM((_CH,), jnp.int32),
        pltpu.VMEM((_CH,), jnp.float32),
        pltpu.VMEM((_CH,), jnp.int32),
        pltpu.VMEM((_CH,), jnp.float32),
        pltpu.VMEM((_CH,), jnp.float32),
        pltpu.VMEM((_CH,), jnp.int32),
        pltpu.VMEM((_CH,), jnp.float32),
        pltpu.VMEM((_CH,), jnp.float32),
        pltpu.VMEM((_CH,), jnp.int32),
        pltpu.VMEM((_CH,), jnp.float32),
        pltpu.VMEM((_CH,), jnp.float32),
        pltpu.VMEM((_CH,), jnp.float32),
        pltpu.VMEM((_CH,), jnp.float32),
        pltpu.VMEM((16,), jnp.float32),
        pltpu.VMEM((16,), jnp.float32),
        pltpu.VMEM((16,), jnp.float32),
        pltpu.VMEM((16,), jnp.float32),
        pltpu.VMEM((_NT, 16), jnp.float32),
        pltpu.VMEM((16,), jnp.float32),
        pltpu.VMEM((_NT, 16), jnp.float32),
        pltpu.VMEM((_CH,), jnp.int32),
        pltpu.VMEM((_CH,), jnp.float32),
        pltpu.VMEM((_CH,), jnp.float32),
        pltpu.VMEM_SHARED((_NT, 16), jnp.float32),
        pltpu.VMEM_SHARED((_NT, 16), jnp.float32),
        pltpu.VMEM_SHARED((_DATA_LEN,), jnp.float32),
        pltpu.SemaphoreType.DMA,
        pltpu.SemaphoreType.DMA,
        pltpu.SemaphoreType.DMA,
        pltpu.SemaphoreType.DMA,
        pltpu.SemaphoreType.DMA,
        pltpu.SemaphoreType.DMA,
    ],
)


def kernel(sn, sd, sn_buf, sd_buf, a, b, alpha, y_true, ids):
    sn_f = sn.reshape(_B)
    sd_f = sd.reshape(_B)
    out = _kern(
        sn_f, sd_f,
        sn_buf.reshape(_DATA_LEN), sd_buf.reshape(_DATA_LEN),
        ids.reshape(_NT, _CH),
        sn_f.reshape(_NT, _CH), sd_f.reshape(_NT, _CH),
        y_true.reshape(_NT, _CH),
        jnp.broadcast_to(a, (16,)), jnp.broadcast_to(b, (16,)),
        jnp.broadcast_to(alpha, (16,)),
    )
    return out[0]


# speedup vs baseline: 1.0211x; 1.0211x over previous
import jax
import jax.numpy as jnp
from jax import lax
from jax.experimental import pallas as pl
from jax.experimental.pallas import tpu as pltpu
from jax.experimental.pallas import tpu_sc as plsc

_DATA_LEN = 1000000
_B = 16384
_G1 = 0.9
_G2 = 0.9
_NT = 16
_CH = _B // _NT
_NR = _CH // 16
_K_FAST = 3


def _scal(w):
    lane = lax.iota(jnp.int32, 16)
    return (w + lane.astype(w.dtype) * 0.0)[0]


def _lsum(w):
    lane = lax.iota(jnp.int32, 16)
    for step in (8, 4, 2, 1):
        w = w + jnp.take(w, lane ^ step)
    return w


def _body(sn_hbm, sd_hbm, snb_hbm, sdb_hbm, idst_hbm,
          snt_hbm, sdt_hbm, yt_hbm, a_hbm, b_hbm, al_hbm,
          out_hbm,
          ids_f, ivf_f, iv_f, sv_f, gf_f, g_f,
          sn_v, sd_v, y_v, bufn, bufd, snw, sdw,
          a_v, b_v, al_v, pub, allc, psum, allp, pm_f, candv_f, cgat_f,
          spm_i, spm_f, w_spm,
          sem1, sem2, sem3, sem4, sem5, sem6):
    c = lax.axis_index("c")
    tid = lax.axis_index("s")

    @pl.when(c == 0)
    def _core0():
        lane = lax.iota(jnp.int32, 16)
        base = tid * _CH

        pltpu.sync_copy(idst_hbm.at[tid], ids_f)
        h_bn = pltpu.async_copy(snb_hbm.at[ids_f], bufn, sem1)
        h_bd = pltpu.async_copy(sdb_hbm.at[ids_f], bufd, sem2)
        h_sn = pltpu.async_copy(snt_hbm.at[tid], sn_v, sem3)
        h_sd = pltpu.async_copy(sdt_hbm.at[tid], sd_v, sem4)
        h_y = pltpu.async_copy(yt_hbm.at[tid], y_v, sem5)
        h_a = pltpu.async_copy(a_hbm, a_v, sem6)
        h_b = pltpu.async_copy(b_hbm, b_v, sem6)
        h_al = pltpu.async_copy(al_hbm, al_v, sem6)

        for r in range(_NR):
            sl = pl.ds(r * 16, 16)
            iv_f[sl] = base + r * 16 + lane
            ivf_f[sl] = (base + r * 16 + lane).astype(jnp.float32)

        pltpu.sync_copy(ivf_f, w_spm.at[ids_f])
        plsc.subcore_barrier()

        for _rr in range(_K_FAST):
            pltpu.sync_copy(w_spm.at[ids_f], gf_f)
            plsc.subcore_barrier()
            for r in range(_NR):
                sl = pl.ds(r * 16, 16)
                sv_f[sl] = jnp.maximum(ivf_f[sl], gf_f[sl])
            pltpu.sync_copy(sv_f, w_spm.at[ids_f])
            plsc.subcore_barrier()

        pltpu.sync_copy(w_spm.at[ids_f], gf_f)
        nv = jnp.zeros((16,), jnp.float32)
        for r in range(_NR):
            sl = pl.ds(r * 16, 16)
            nv = nv + jnp.where(ivf_f[sl] > gf_f[sl], 1.0, 0.0)
        pub[...] = nv
        pltpu.sync_copy(pub, spm_i.at[tid])
        plsc.subcore_barrier()
        pltpu.sync_copy(spm_i, allc)
        tot = jnp.zeros((16,), jnp.float32)
        for t in range(_NT):
            tot = tot + allc[t]
        need_slow = _scal(_lsum(tot)) > 0.5

        for r in range(_NR):
            sl = pl.ds(r * 16, 16)
            g_f[sl] = gf_f[sl].astype(jnp.int32)

        @pl.when(need_slow)
        def _bit_descent():
            for r in range(_NR):
                sl = pl.ds(r * 16, 16)
                pm_f[sl] = jnp.zeros((16,), jnp.int32) + 1
                g_f[sl] = jnp.zeros((16,), jnp.int32)

            def _bit_round(bnum, carry):
                bitpos = 13 - bnum
                for r in range(_NR):
                    sl = pl.ds(r * 16, 16)
                    candv_f[sl] = jnp.zeros((16,), jnp.float32)
                pltpu.sync_copy(candv_f, w_spm.at[ids_f])
                plsc.subcore_barrier()
                for r in range(_NR):
                    sl = pl.ds(r * 16, 16)
                    bitv = lax.shift_right_logical(iv_f[sl], bitpos) & 1
                    candv_f[sl] = (pm_f[sl] & bitv).astype(jnp.float32)
                pltpu.sync_copy(candv_f, w_spm.at[ids_f], add=True)
                plsc.subcore_barrier()
                pltpu.sync_copy(w_spm.at[ids_f], cgat_f)
                for r in range(_NR):
                    sl = pl.ds(r * 16, 16)
                    mb = jnp.where(cgat_f[sl] > 0.5, 1, 0)
                    bitv = lax.shift_right_logical(iv_f[sl], bitpos) & 1
                    g_f[sl] = g_f[sl] | lax.shift_left(mb, bitpos)
                    pm_f[sl] = pm_f[sl] & jnp.where(bitv == mb, 1, 0)
                plsc.subcore_barrier()
                return carry

            lax.fori_loop(0, 14, _bit_round, jnp.int32(0))

        h_sw = pltpu.async_copy(sn_hbm.at[g_f], snw, sem1)
        h_dw = pltpu.async_copy(sd_hbm.at[g_f], sdw, sem2)
        h_sn.wait()
        h_sd.wait()
        h_y.wait()
        h_a.wait()
        h_b.wait()
        h_al.wait()
        h_bn.wait()
        h_bd.wait()
        h_sw.wait()
        h_dw.wait()

        av = a_v[...]
        bv = b_v[...]
        z = jnp.zeros((16,), jnp.float32)
        acc_gwp = z
        acc_gwn = z
        acc_ap = z
        acc_an = z
        acc_ga = z
        acc_gb = z
        acc_cp = z
        acc_cn = z
        for r in range(_NR):
            sl = pl.ds(r * 16, 16)
            vsn = (1.0 - _G1) * bufn[sl] + _G1 * snw[sl]
            vsd = (1.0 - _G2) * bufd[sl] + _G2 * sdw[sl]
            vsd = jnp.maximum(vsd, 1e-8)
            snd = 1.0 / (1.0 + jnp.exp(-(vsn / vsd)))
            gsnd = snd * (1.0 - snd)
            gw = gsnd * (sn_v[sl] / vsd - vsn / (vsd * vsd) * sd_v[sl])
            mp = jnp.where(y_v[sl] == 1, 1.0, 0.0)
            mn = 1.0 - mp
            dp = snd - av
            dn = snd - bv
            acc_gwp = acc_gwp + mp * (2.0 * dp * gw)
            acc_gwn = acc_gwn + mn * (2.0 * dn * gw)
            acc_ap = acc_ap + mp * gw
            acc_an = acc_an + mn * gw
            acc_ga = acc_ga + mp * dp * dp
            acc_gb = acc_gb + mn * dn * dn
            acc_cp = acc_cp + mp
            acc_cn = acc_cn + mn

        sums = (acc_gwp, acc_gwn, acc_ap, acc_an,
                acc_ga, acc_gb, acc_cp, acc_cn)
        vec = z
        for k, acc in enumerate(sums):
            vec = jnp.where(lane == k, _lsum(acc), vec)
        psum[...] = vec
        pltpu.sync_copy(psum, spm_f.at[tid])
        plsc.subcore_barrier()

        @pl.when(tid == 0)
        def _final():
            pltpu.sync_copy(spm_f, allp)
            tot2 = z
            for t in range(_NT):
                tot2 = tot2 + allp[t]
            s_gwp, s_gwn, s_ap, s_an, s_ga, s_gb, s_cp, s_cn = (
                jnp.take(tot2, lane * 0 + k) for k in range(8))
            alpha_v = al_v[...]
            loss = (s_gwp / s_cp + s_gwn / s_cn
                    + alpha_v * (s_an / s_cn - s_ap / s_cp)
                    + s_ga / s_cp + s_gb / s_cn)
            psum[...] = jnp.where(lane == 0, loss, 0.0)
            pltpu.sync_copy(psum, out_hbm)


_mesh = plsc.VectorSubcoreMesh(core_axis_name="c", subcore_axis_name="s")

_kern = pl.kernel(
    _body,
    out_type=jax.ShapeDtypeStruct((16,), jnp.float32),
    mesh=_mesh,
    scratch_types=[
        pltpu.VMEM((_CH,), jnp.int32),
        pltpu.VMEM((_CH,), jnp.float32),
        pltpu.VMEM((_CH,), jnp.int32),
        pltpu.VMEM((_CH,), jnp.float32),
        pltpu.VMEM((_CH,), jnp.float32),
        pltpu.VMEM((_CH,), jnp.int32),
        pltpu.VMEM((_CH,), jnp.float32),
        pltpu.VMEM((_CH,), jnp.float32),
        pltpu.VMEM((_CH,), jnp.int32),
        pltpu.VMEM((_CH,), jnp.float32),
        pltpu.VMEM((_CH,), jnp.float32),
        pltpu.VMEM((_CH,), jnp.float32),
        pltpu.VMEM((_CH,), jnp.float32),
        pltpu.VMEM((16,), jnp.float32),
        pltpu.VMEM((16,), jnp.float32),
        pltpu.VMEM((16,), jnp.float32),
        pltpu.VMEM((16,), jnp.float32),
        pltpu.VMEM((_NT, 16), jnp.float32),
        pltpu.VMEM((16,), jnp.float32),
        pltpu.VMEM((_NT, 16), jnp.float32),
        pltpu.VMEM((_CH,), jnp.int32),
        pltpu.VMEM((_CH,), jnp.float32),
        pltpu.VMEM((_CH,), jnp.float32),
        pltpu.VMEM_SHARED((_NT, 16), jnp.float32),
        pltpu.VMEM_SHARED((_NT, 16), jnp.float32),
        pltpu.VMEM_SHARED((_DATA_LEN,), jnp.float32),
        pltpu.SemaphoreType.DMA,
        pltpu.SemaphoreType.DMA,
        pltpu.SemaphoreType.DMA,
        pltpu.SemaphoreType.DMA,
        pltpu.SemaphoreType.DMA,
        pltpu.SemaphoreType.DMA,
    ],
)


def kernel(sn, sd, sn_buf, sd_buf, a, b, alpha, y_true, ids):
    sn_f = sn.reshape(_B)
    sd_f = sd.reshape(_B)
    out = _kern(
        sn_f, sd_f,
        sn_buf.reshape(_DATA_LEN), sd_buf.reshape(_DATA_LEN),
        ids.reshape(_NT, _CH),
        sn_f.reshape(_NT, _CH), sd_f.reshape(_NT, _CH),
        y_true.reshape(_NT, _CH),
        jnp.broadcast_to(a, (16,)), jnp.broadcast_to(b, (16,)),
        jnp.broadcast_to(alpha, (16,)),
    )
    return out[0]
